# Initial kernel scaffold; baseline (speedup 1.0000x reference)
import jax
import jax.numpy as jnp
from jax.experimental import pallas as pl


def kernel(spans, table):
    raise NotImplementedError("write your pallas kernel here")


# speedup vs baseline: 1.0372x; 1.0372x over previous
import functools

import jax
import jax.numpy as jnp
from jax import lax
from jax.experimental import pallas as pl
from jax.experimental.pallas import tpu as pltpu
from jax.experimental.pallas import tpu_sc as plsc

NC = 2
NS = 16
NW = NC * NS


def _span_sum_body(spans_hbm, table_hbm, out_hbm, idx_v, acc_v, sem):
    wid = lax.axis_index("s") * NC + lax.axis_index("c")
    L = idx_v.shape[0]
    bpw = acc_v.shape[0]

    pltpu.sync_copy(spans_hbm.at[wid], idx_v)

    pltpu.async_copy(table_hbm.at[idx_v.at[0]], acc_v, sem).wait()

    @pl.loop(1, L)
    def _fire(l):
        pltpu.async_copy(table_hbm.at[idx_v.at[l]], acc_v, sem, add=True)

    @pl.loop(1, L)
    def _drain(l):
        del l
        pltpu.make_async_copy(table_hbm.at[idx_v.at[0]], acc_v, sem).wait()

    pltpu.sync_copy(acc_v, out_hbm.at[pl.ds(wid * bpw, bpw)])


def kernel(spans, table):
    B, L = spans.shape
    V, D = table.shape
    bpw = B // NW
    spans_r = spans.astype(jnp.int32).T.reshape(L, NW, bpw).transpose(1, 0, 2)

    mesh = plsc.VectorSubcoreMesh(
        core_axis_name="c", subcore_axis_name="s", num_cores=NC, num_subcores=NS
    )
    f = pl.kernel(
        _span_sum_body,
        out_type=jax.ShapeDtypeStruct((B, D), jnp.float32),
        mesh=mesh,
        scratch_types=[
            pltpu.VMEM((L, bpw), jnp.int32),
            pltpu.VMEM((bpw, D), jnp.float32),
            pltpu.SemaphoreType.DMA,
        ],
        compiler_params=pltpu.CompilerParams(use_tc_tiling_on_sc=False),
    )
    return f(spans_r, table)
